# baseline (speedup 1.0000x reference)
import functools

import jax
import jax.numpy as jnp
from jax import lax
from jax.experimental import pallas as pl
from jax.experimental.pallas import tpu as pltpu
from jax.experimental.pallas import tpu_sc as plsc

NC = 2
NS = 16
LANES = 16
NW = NC * NS

CH = 80


def _sc_spmm(emb, pk, w, zeros):
    N, D = emb.shape
    n_chunks = pk.shape[1]
    epw = n_chunks * CH
    Np = zeros.shape[0]
    rows_per_tile = Np // NS

    assert n_chunks % 3 == 2 and n_chunks >= 8

    mesh = plsc.VectorSubcoreMesh(core_axis_name="c", subcore_axis_name="s")

    @functools.partial(
        pl.kernel,
        out_type=jax.ShapeDtypeStruct((NC, Np, D), jnp.float32),
        mesh=mesh,
        scratch_types=[
            pltpu.VMEM_SHARED((Np, D), jnp.float32),
            pltpu.VMEM((epw,), jnp.float32),
            pltpu.VMEM((2, CH), jnp.int32),
            pltpu.VMEM((2, CH), jnp.int32),
            pltpu.VMEM((2, CH), jnp.int32),
            pltpu.VMEM((CH,), jnp.int32),
            pltpu.VMEM((CH,), jnp.int32),
            pltpu.VMEM((CH,), jnp.int32),
            pltpu.VMEM((CH, D), jnp.float32),
            pltpu.VMEM((CH, D), jnp.float32),
            pltpu.VMEM((CH, D), jnp.float32),
            pltpu.SemaphoreType.DMA,
            pltpu.SemaphoreType.DMA,
            pltpu.SemaphoreType.DMA,
            pltpu.SemaphoreType.DMA,
            pltpu.SemaphoreType.DMA,
            pltpu.SemaphoreType.DMA,
            pltpu.SemaphoreType.DMA,
            pltpu.SemaphoreType.DMA,
            pltpu.SemaphoreType.DMA,
        ],
    )
    def spmm(emb_hbm, pk_hbm, w_hbm, zeros_hbm, parts_hbm,
             agg_sh, w_v, eb0, eb1, eb2, sd0, sd1, sd2, r0b, r1b, r2b,
             es0, es1, es2, gs0, gs1, gs2, ss0, ss1, ss2):
        c = lax.axis_index("c")
        s = lax.axis_index("s")
        wid = s * NC + c
        row0 = s * rows_per_tile
        pltpu.sync_copy(zeros_hbm.at[pl.ds(row0, rows_per_tile)],
                        agg_sh.at[pl.ds(row0, rows_per_tile)])
        pltpu.sync_copy(w_hbm.at[wid], w_v)
        plsc.subcore_barrier()

        ebufs = (eb0, eb1, eb2)
        sdsts = (sd0, sd1, sd2)
        rows = (r0b, r1b, r2b)
        esems = (es0, es1, es2)
        gsems = (gs0, gs1, gs2)
        ssems = (ss0, ss1, ss2)

        def wait_gather(b):
            pltpu.make_async_copy(emb_hbm.at[ebufs[b].at[0]],
                                  rows[b], gsems[b]).wait()

        def drain_scatter(b):
            pltpu.make_async_copy(rows[b], agg_sh.at[sdsts[b]],
                                  ssems[b]).wait()

        def mul_chunk(i, b):
            rbuf = rows[b]

            @plsc.parallel_loop(0, CH // LANES, unroll=2)
            def _(g):
                w16 = w_v[pl.ds(i * CH + g * LANES, LANES)]
                for el in range(LANES):
                    wb = w16[el]
                    e = g * LANES + el
                    for k in range(D // LANES):
                        sl = pl.ds(k * LANES, LANES)
                        rbuf[e, sl] = rbuf[e, sl] * wb

        def iteration(i, b, wait_ssem=True, issue_eload=True, issue_next=True):
            b1 = (b + 1) % 3
            wait_gather(b)
            mul_chunk(i, b)
            for j in range(CH // LANES):
                sl = pl.ds(j * LANES, LANES)
                sdsts[b][sl] = ebufs[b][1, sl]
            if issue_eload:
                pltpu.async_copy(pk_hbm.at[wid, i + 3], ebufs[b], esems[b])
            pltpu.async_copy(rows[b], agg_sh.at[sdsts[b]], ssems[b], add=True)
            if wait_ssem:
                drain_scatter(b1)
            if issue_next:
                pltpu.make_async_copy(pk_hbm.at[wid, 0],
                                      ebufs[b1], esems[b1]).wait()
                pltpu.async_copy(emb_hbm.at[ebufs[b1].at[0]],
                                 rows[b1], gsems[b1])

        pltpu.sync_copy(pk_hbm.at[wid, 0], eb0)
        pltpu.async_copy(pk_hbm.at[wid, 1], eb1, es1)
        pltpu.async_copy(pk_hbm.at[wid, 2], eb2, es2)
        pltpu.async_copy(emb_hbm.at[eb0.at[0]], r0b, gs0)

        iteration(0, 0, wait_ssem=False)
        iteration(1, 1, wait_ssem=False)

        @pl.loop(0, (n_chunks - 5) // 3)
        def _(t):
            base = 3 * t + 2
            iteration(base, 2)
            iteration(base + 1, 0)
            iteration(base + 2, 1)

        iteration(n_chunks - 3, 2, issue_eload=False)
        iteration(n_chunks - 2, 0, issue_eload=False)
        iteration(n_chunks - 1, 1, issue_eload=False, issue_next=False)
        drain_scatter(0)
        drain_scatter(1)

        plsc.subcore_barrier()
        pltpu.sync_copy(agg_sh.at[pl.ds(row0, rows_per_tile)],
                        parts_hbm.at[c, pl.ds(row0, rows_per_tile)])

    return spmm(emb, pk, w, zeros)


def _tc_dense(emb, parts, W1, b1, W2, b2):
    N, D = emb.shape
    BM = 2000
    dn = (((1,), (1,)), ((), ()))

    def body(emb_ref, parts_ref, w1_ref, b1_ref, w2_ref, b2_ref, out_ref):
        x = emb_ref[...]
        agg = parts_ref[0] + parts_ref[1]
        w1 = w1_ref[...]
        w2 = w2_ref[...]
        b1v = b1_ref[...]
        b2v = b2_ref[...]
        self_emb = lax.dot_general(x, w1, dn, preferred_element_type=jnp.float32) + b1v
        neigh = lax.dot_general(agg, w2, dn, preferred_element_type=jnp.float32) + b2v
        inter = lax.dot_general(neigh * x, w2, dn,
                                preferred_element_type=jnp.float32) + b2v
        o = self_emb + neigh + inter
        out_ref[...] = jnp.where(o >= 0, o, 0.2 * o)

    return pl.pallas_call(
        body,
        grid=(N // BM,),
        in_specs=[
            pl.BlockSpec((BM, D), lambda i: (i, 0)),
            pl.BlockSpec((NC, BM, D), lambda i: (0, i, 0)),
            pl.BlockSpec((D, D), lambda i: (0, 0)),
            pl.BlockSpec((1, D), lambda i: (0, 0)),
            pl.BlockSpec((D, D), lambda i: (0, 0)),
            pl.BlockSpec((1, D), lambda i: (0, 0)),
        ],
        out_specs=pl.BlockSpec((BM, D), lambda i: (i, 0)),
        out_shape=jax.ShapeDtypeStruct((N, D), jnp.float32),
    )(emb, parts, W1, b1.reshape(1, D), W2, b2.reshape(1, D))


def kernel(embeddings, adj_edge_index, adj_edge_weight, W1, b1, W2, b2):
    N, D = embeddings.shape
    E = adj_edge_index.shape[1]
    epw = E // NW
    n_chunks = epw // CH
    Np = -(-N // (8 * NS)) * (8 * NS)
    pk = jnp.stack([adj_edge_index[0].reshape(NW, n_chunks, CH),
                    adj_edge_index[1].reshape(NW, n_chunks, CH)], axis=2)
    zeros = jnp.zeros((Np, D), embeddings.dtype)
    parts = _sc_spmm(embeddings, pk, adj_edge_weight.reshape(NW, epw), zeros)
    return _tc_dense(embeddings, parts, W1, b1, W2, b2)
